# baseline (speedup 1.0000x reference)
import functools

import jax
import jax.numpy as jnp
from jax import lax
from jax.experimental import pallas as pl
from jax.experimental.pallas import tpu as pltpu
from jax.experimental.pallas import tpu_sc as plsc

N_ROWS = 2_000_000
C = 10
NC, NS = 2, 16
NW = NC * NS
BASE = (N_ROWS // (NW * 128)) * 128
TAIL = N_ROWS - NW * BASE
TSTART = NW * BASE
NTW = TAIL // 128
R = 1024
NCH = BASE // R
NPAIR = NCH // 2
G = R // 16

LOWERS = (0.0, 0.06666667014360428, 0.13333334028720856, 0.20000001788139343,
          0.2666666805744171, 0.3333333432674408, 0.40000003576278687,
          0.46666669845581055, 0.5333333611488342, 0.6000000238418579,
          0.6666666865348816, 0.7333333492279053, 0.8000000715255737,
          0.8666667342185974, 0.9333333969116211)

_mesh = plsc.VectorSubcoreMesh(core_axis_name="c", subcore_axis_name="s",
                               num_cores=NC, num_subcores=NS)
_params = pltpu.CompilerParams(needs_layout_passes=False,
                               use_tc_tiling_on_sc=True)


@functools.partial(
    pl.kernel,
    out_type=jax.ShapeDtypeStruct((NW * 64,), jnp.float32),
    mesh=_mesh,
    scratch_types=[
        pltpu.VMEM((C, R), jnp.float32),
        pltpu.VMEM((C, R), jnp.float32),
        pltpu.VMEM((R,), jnp.int32),
        pltpu.VMEM((R,), jnp.int32),
        pltpu.VMEM((C, 128), jnp.float32),
        pltpu.VMEM((128,), jnp.int32),
        pltpu.VMEM((256,), jnp.float32),
        pltpu.VMEM((256,), jnp.float32),
        pltpu.VMEM((256,), jnp.float32),
        pltpu.VMEM((16,), jnp.float32),
        pltpu.VMEM((64,), jnp.float32),
        pltpu.SemaphoreType.DMA,
        pltpu.SemaphoreType.DMA,
        pltpu.SemaphoreType.DMA,
        pltpu.SemaphoreType.DMA,
    ],
    compiler_params=_params,
)
def _partials_kernel(pt_hbm, labels_hbm, out_hbm,
                     pbuf0, pbuf1, lbuf0, lbuf1, tbuf, tlbuf,
                     cnt_acc, conf_acc, acc_acc, bri_acc, partial,
                     psem0, psem1, lsem0, lsem1):
    wid = lax.axis_index("s") * NC + lax.axis_index("c")
    row0 = wid * BASE
    lane = lax.iota(jnp.int32, 16)
    zeros = jnp.zeros((16,), jnp.float32)
    ones = jnp.ones((16,), jnp.float32)

    for off in range(0, 256, 16):
        cnt_acc[pl.ds(off, 16)] = zeros
        conf_acc[pl.ds(off, 16)] = zeros
        acc_acc[pl.ds(off, 16)] = zeros
    bri_acc[...] = zeros

    def issue(pb, lb, ps, ls, ci):
        r0 = pl.multiple_of(row0 + ci * R, 128)
        pltpu.make_async_copy(pt_hbm.at[:, pl.ds(r0, R)], pb, ps).start()
        pltpu.make_async_copy(labels_hbm.at[pl.ds(r0, R)], lb, ls).start()

    def wait(pb, lb, ps, ls, ci):
        r0 = pl.multiple_of(row0 + ci * R, 128)
        pltpu.make_async_copy(pt_hbm.at[:, pl.ds(r0, R)], pb, ps).wait()
        pltpu.make_async_copy(labels_hbm.at[pl.ds(r0, R)], lb, ls).wait()

    def consume(pb, lb, ngroups):
        def body(g, carry):
            o = pl.multiple_of(g * 16, 16)
            rows = g * 16 + lane
            lab = lb[pl.ds(o, 16)]
            v = pb[0, pl.ds(o, 16)]
            s = v
            m = v
            sq = v * v
            for c in range(1, C):
                v = pb[c, pl.ds(o, 16)]
                s = s + v
                m = jnp.maximum(m, v)
                sq = sq + v * v
            sel = plsc.load_gather(pb, [lab, rows])
            inv = 1.0 / s
            conf = m * inv
            accv = (sel == m).astype(jnp.float32)
            bri_acc[...] = bri_acc[...] + (sq * inv * inv - 2.0 * sel * inv + 1.0)
            b = jnp.minimum((conf * 15.0).astype(jnp.int32), 14)
            b = b - (conf <= b.astype(jnp.float32) * LOWERS[1]).astype(jnp.int32)
            b = b + (conf > (b + 1).astype(jnp.float32) * LOWERS[1]).astype(jnp.int32)
            b = jnp.minimum(b, 14)
            idx = lane * 16 + b
            plsc.addupdate_scatter(cnt_acc, [idx], ones)
            plsc.addupdate_scatter(conf_acc, [idx], conf)
            plsc.addupdate_scatter(acc_acc, [idx], accv)
            return carry
        lax.fori_loop(0, ngroups, body, 0)

    issue(pbuf0, lbuf0, psem0, lsem0, 0)

    def pair(j, carry):
        c0 = 2 * j
        issue(pbuf1, lbuf1, psem1, lsem1, c0 + 1)
        wait(pbuf0, lbuf0, psem0, lsem0, c0)
        consume(pbuf0, lbuf0, G)
        issue(pbuf0, lbuf0, psem0, lsem0, c0 + 2)
        wait(pbuf1, lbuf1, psem1, lsem1, c0 + 1)
        consume(pbuf1, lbuf1, G)
        return carry

    lax.fori_loop(0, NPAIR, pair, 0)
    wait(pbuf0, lbuf0, psem0, lsem0, NCH - 1)
    consume(pbuf0, lbuf0, G)

    if TAIL:
        @pl.when(wid < NTW)
        def _():
            t0 = pl.multiple_of(TSTART + wid * 128, 128)
            pltpu.make_async_copy(pt_hbm.at[:, pl.ds(t0, 128)], tbuf,
                                  psem1).start()
            pltpu.make_async_copy(labels_hbm.at[pl.ds(t0, 128)], tlbuf,
                                  lsem1).start()
            pltpu.make_async_copy(pt_hbm.at[:, pl.ds(t0, 128)], tbuf,
                                  psem1).wait()
            pltpu.make_async_copy(labels_hbm.at[pl.ds(t0, 128)], tlbuf,
                                  lsem1).wait()
            consume(tbuf, tlbuf, 8)

    cnt = cnt_acc[pl.ds(0, 16)]
    cf = conf_acc[pl.ds(0, 16)]
    ac = acc_acc[pl.ds(0, 16)]
    for l in range(1, 16):
        cnt = cnt + cnt_acc[pl.ds(16 * l, 16)]
        cf = cf + conf_acc[pl.ds(16 * l, 16)]
        ac = ac + acc_acc[pl.ds(16 * l, 16)]
    partial[pl.ds(0, 16)] = cnt
    partial[pl.ds(16, 16)] = cf
    partial[pl.ds(32, 16)] = ac
    partial[pl.ds(48, 16)] = bri_acc[...]
    off = pl.multiple_of(wid * 64, 8)
    pltpu.sync_copy(partial, out_hbm.at[pl.ds(off, 64)])


@functools.partial(
    pl.kernel,
    out_type=jax.ShapeDtypeStruct((16,), jnp.float32),
    mesh=_mesh,
    scratch_types=[
        pltpu.VMEM((NW * 64,), jnp.float32),
        pltpu.VMEM((16,), jnp.float32),
    ],
    compiler_params=_params,
)
def _finalize_kernel(partials_hbm, out_hbm, buf, obuf):
    wid = lax.axis_index("s") * NC + lax.axis_index("c")

    @pl.when(wid == 0)
    def _():
        pltpu.sync_copy(partials_hbm, buf)
        cnt = buf[pl.ds(0, 16)]
        cf = buf[pl.ds(16, 16)]
        ac = buf[pl.ds(32, 16)]
        bri = buf[pl.ds(48, 16)]
        for w in range(1, NW):
            o = w * 64
            cnt = cnt + buf[pl.ds(o, 16)]
            cf = cf + buf[pl.ds(o + 16, 16)]
            ac = ac + buf[pl.ds(o + 32, 16)]
            bri = bri + buf[pl.ds(o + 48, 16)]
        n = jnp.float32(N_ROWS)
        prop = cnt / n
        safe = jnp.maximum(cnt, 1.0)
        gap = jnp.abs(cf / safe - ac / safe)
        gapv = jnp.where(cnt > 0.0, gap, 0.0)
        ece = jnp.sum(gapv * prop)
        max_err = jnp.max(gapv)
        brim = jnp.sum(bri / n)
        lane = lax.iota(jnp.int32, 16)
        obuf[...] = (jnp.where(lane == 0, ece, 0.0)
                     + jnp.where(lane == 1, max_err, 0.0)
                     + jnp.where(lane == 2, brim, 0.0))
        pltpu.sync_copy(obuf, out_hbm)


def kernel(probs, labels):
    pt = probs.T
    lab = labels.astype(jnp.int32)
    partials = _partials_kernel(pt, lab)
    res = _finalize_kernel(partials)
    return (res[0:1], res[1:2], res[2])


# speedup vs baseline: 1.4288x; 1.4288x over previous
import functools

import jax
import jax.numpy as jnp
from jax import lax
from jax.experimental import pallas as pl
from jax.experimental.pallas import tpu as pltpu
from jax.experimental.pallas import tpu_sc as plsc

N_ROWS = 2_000_000
C = 10
NC, NS = 2, 16
NW = NC * NS
BASE = (N_ROWS // (NW * 128)) * 128
TAIL = N_ROWS - NW * BASE
TSTART = NW * BASE
NTW = TAIL // 128
R = 1024
NCH = BASE // R
NPAIR = NCH // 2
G = R // 16

LOWERS = (0.0, 0.06666667014360428, 0.13333334028720856, 0.20000001788139343,
          0.2666666805744171, 0.3333333432674408, 0.40000003576278687,
          0.46666669845581055, 0.5333333611488342, 0.6000000238418579,
          0.6666666865348816, 0.7333333492279053, 0.8000000715255737,
          0.8666667342185974, 0.9333333969116211)

_mesh = plsc.VectorSubcoreMesh(core_axis_name="c", subcore_axis_name="s",
                               num_cores=NC, num_subcores=NS)
_params = pltpu.CompilerParams(needs_layout_passes=False,
                               use_tc_tiling_on_sc=True)


@functools.partial(
    pl.kernel,
    out_type=jax.ShapeDtypeStruct((NW * 64,), jnp.float32),
    mesh=_mesh,
    scratch_types=[
        pltpu.VMEM((C, R), jnp.float32),
        pltpu.VMEM((C, R), jnp.float32),
        pltpu.VMEM((R,), jnp.int32),
        pltpu.VMEM((R,), jnp.int32),
        pltpu.VMEM((C, 128), jnp.float32),
        pltpu.VMEM((128,), jnp.int32),
        pltpu.VMEM((256,), jnp.float32),
        pltpu.VMEM((256,), jnp.float32),
        pltpu.VMEM((256,), jnp.float32),
        pltpu.VMEM((16,), jnp.float32),
        pltpu.VMEM((64,), jnp.float32),
        pltpu.SemaphoreType.DMA,
        pltpu.SemaphoreType.DMA,
        pltpu.SemaphoreType.DMA,
        pltpu.SemaphoreType.DMA,
    ],
    compiler_params=_params,
)
def _partials_kernel(pt_hbm, labels_hbm, out_hbm,
                     pbuf0, pbuf1, lbuf0, lbuf1, tbuf, tlbuf,
                     cnt_acc, conf_acc, acc_acc, bri_acc, partial,
                     psem0, psem1, lsem0, lsem1):
    wid = lax.axis_index("s") * NC + lax.axis_index("c")
    row0 = wid * BASE
    lane = lax.iota(jnp.int32, 16)
    zeros = jnp.zeros((16,), jnp.float32)
    ones = jnp.ones((16,), jnp.float32)

    for off in range(0, 256, 16):
        cnt_acc[pl.ds(off, 16)] = zeros
        conf_acc[pl.ds(off, 16)] = zeros
        acc_acc[pl.ds(off, 16)] = zeros
    bri_acc[...] = zeros

    def issue(pb, lb, ps, ls, ci):
        r0 = pl.multiple_of(row0 + ci * R, 128)
        pltpu.make_async_copy(pt_hbm.at[:, pl.ds(r0, R)], pb, ps).start()
        pltpu.make_async_copy(labels_hbm.at[pl.ds(r0, R)], lb, ls).start()

    def wait(pb, lb, ps, ls, ci):
        r0 = pl.multiple_of(row0 + ci * R, 128)
        pltpu.make_async_copy(pt_hbm.at[:, pl.ds(r0, R)], pb, ps).wait()
        pltpu.make_async_copy(labels_hbm.at[pl.ds(r0, R)], lb, ls).wait()

    def consume(pb, lb, ngroups):
        def group(o):
            rows = o + lane
            lab = lb[pl.ds(o, 16)]
            v = pb[0, pl.ds(o, 16)]
            s = v
            m = v
            sq = v * v
            for c in range(1, C):
                v = pb[c, pl.ds(o, 16)]
                s = s + v
                m = jnp.maximum(m, v)
                sq = sq + v * v
            sel = plsc.load_gather(pb, [lab, rows])
            inv = 1.0 / s
            conf = m * inv
            accv = (sel == m).astype(jnp.float32)
            bri_t = sq * inv * inv - 2.0 * sel * inv + 1.0
            b = (conf > LOWERS[1]).astype(jnp.int32)
            for k in range(2, 15):
                b = b + (conf > LOWERS[k]).astype(jnp.int32)
            return conf, accv, bri_t, lane * 16 + b

        def body(j, carry):
            o = pl.multiple_of(j * 32, 32)
            conf0, acc0, bri0, idx0 = group(o)
            conf1, acc1, bri1, idx1 = group(o + 16)
            bri_acc[...] = bri_acc[...] + (bri0 + bri1)
            plsc.addupdate_scatter(cnt_acc, [idx0], ones)
            plsc.addupdate_scatter(conf_acc, [idx0], conf0)
            plsc.addupdate_scatter(acc_acc, [idx0], acc0)
            plsc.addupdate_scatter(cnt_acc, [idx1], ones)
            plsc.addupdate_scatter(conf_acc, [idx1], conf1)
            plsc.addupdate_scatter(acc_acc, [idx1], acc1)
            return carry
        lax.fori_loop(0, ngroups // 2, body, 0)

    issue(pbuf0, lbuf0, psem0, lsem0, 0)

    def pair(j, carry):
        c0 = 2 * j
        issue(pbuf1, lbuf1, psem1, lsem1, c0 + 1)
        wait(pbuf0, lbuf0, psem0, lsem0, c0)
        consume(pbuf0, lbuf0, G)
        issue(pbuf0, lbuf0, psem0, lsem0, c0 + 2)
        wait(pbuf1, lbuf1, psem1, lsem1, c0 + 1)
        consume(pbuf1, lbuf1, G)
        return carry

    lax.fori_loop(0, NPAIR, pair, 0)
    wait(pbuf0, lbuf0, psem0, lsem0, NCH - 1)
    consume(pbuf0, lbuf0, G)

    if TAIL:
        @pl.when(wid < NTW)
        def _():
            t0 = pl.multiple_of(TSTART + wid * 128, 128)
            pltpu.make_async_copy(pt_hbm.at[:, pl.ds(t0, 128)], tbuf,
                                  psem1).start()
            pltpu.make_async_copy(labels_hbm.at[pl.ds(t0, 128)], tlbuf,
                                  lsem1).start()
            pltpu.make_async_copy(pt_hbm.at[:, pl.ds(t0, 128)], tbuf,
                                  psem1).wait()
            pltpu.make_async_copy(labels_hbm.at[pl.ds(t0, 128)], tlbuf,
                                  lsem1).wait()
            consume(tbuf, tlbuf, 8)

    cnt = cnt_acc[pl.ds(0, 16)]
    cf = conf_acc[pl.ds(0, 16)]
    ac = acc_acc[pl.ds(0, 16)]
    for l in range(1, 16):
        cnt = cnt + cnt_acc[pl.ds(16 * l, 16)]
        cf = cf + conf_acc[pl.ds(16 * l, 16)]
        ac = ac + acc_acc[pl.ds(16 * l, 16)]
    partial[pl.ds(0, 16)] = cnt
    partial[pl.ds(16, 16)] = cf
    partial[pl.ds(32, 16)] = ac
    partial[pl.ds(48, 16)] = bri_acc[...]
    off = pl.multiple_of(wid * 64, 8)
    pltpu.sync_copy(partial, out_hbm.at[pl.ds(off, 64)])


@functools.partial(
    pl.kernel,
    out_type=jax.ShapeDtypeStruct((16,), jnp.float32),
    mesh=_mesh,
    scratch_types=[
        pltpu.VMEM((NW * 64,), jnp.float32),
        pltpu.VMEM((16,), jnp.float32),
    ],
    compiler_params=_params,
)
def _finalize_kernel(partials_hbm, out_hbm, buf, obuf):
    wid = lax.axis_index("s") * NC + lax.axis_index("c")

    @pl.when(wid == 0)
    def _():
        pltpu.sync_copy(partials_hbm, buf)
        cnt = buf[pl.ds(0, 16)]
        cf = buf[pl.ds(16, 16)]
        ac = buf[pl.ds(32, 16)]
        bri = buf[pl.ds(48, 16)]
        for w in range(1, NW):
            o = w * 64
            cnt = cnt + buf[pl.ds(o, 16)]
            cf = cf + buf[pl.ds(o + 16, 16)]
            ac = ac + buf[pl.ds(o + 32, 16)]
            bri = bri + buf[pl.ds(o + 48, 16)]
        n = jnp.float32(N_ROWS)
        prop = cnt / n
        safe = jnp.maximum(cnt, 1.0)
        gap = jnp.abs(cf / safe - ac / safe)
        gapv = jnp.where(cnt > 0.0, gap, 0.0)
        ece = jnp.sum(gapv * prop)
        max_err = jnp.max(gapv)
        brim = jnp.sum(bri / n)
        lane = lax.iota(jnp.int32, 16)
        obuf[...] = (jnp.where(lane == 0, ece, 0.0)
                     + jnp.where(lane == 1, max_err, 0.0)
                     + jnp.where(lane == 2, brim, 0.0))
        pltpu.sync_copy(obuf, out_hbm)


def kernel(probs, labels):
    pt = probs.T
    lab = labels.astype(jnp.int32)
    partials = _partials_kernel(pt, lab)
    res = _finalize_kernel(partials)
    return (res[0:1], res[1:2], res[2])


# speedup vs baseline: 1.4863x; 1.0402x over previous
import functools

import jax
import jax.numpy as jnp
from jax import lax
from jax.experimental import pallas as pl
from jax.experimental.pallas import tpu as pltpu
from jax.experimental.pallas import tpu_sc as plsc

N_ROWS = 2_000_000
C = 10
NC, NS = 2, 16
NW = NC * NS
BASE = (N_ROWS // (NW * 128)) * 128
TAIL = N_ROWS - NW * BASE
TSTART = NW * BASE
NTW = TAIL // 128
R = 1024
NCH = BASE // R
NPAIR = NCH // 2
G = R // 16

LOWERS = (0.0, 0.06666667014360428, 0.13333334028720856, 0.20000001788139343,
          0.2666666805744171, 0.3333333432674408, 0.40000003576278687,
          0.46666669845581055, 0.5333333611488342, 0.6000000238418579,
          0.6666666865348816, 0.7333333492279053, 0.8000000715255737,
          0.8666667342185974, 0.9333333969116211)

_mesh = plsc.VectorSubcoreMesh(core_axis_name="c", subcore_axis_name="s",
                               num_cores=NC, num_subcores=NS)
_params = pltpu.CompilerParams(needs_layout_passes=False,
                               use_tc_tiling_on_sc=True)


@functools.partial(
    pl.kernel,
    out_type=jax.ShapeDtypeStruct((NW * 64,), jnp.float32),
    mesh=_mesh,
    scratch_types=[
        pltpu.VMEM((C, R), jnp.float32),
        pltpu.VMEM((C, R), jnp.float32),
        pltpu.VMEM((R,), jnp.int32),
        pltpu.VMEM((R,), jnp.int32),
        pltpu.VMEM((C, 128), jnp.float32),
        pltpu.VMEM((128,), jnp.int32),
        pltpu.VMEM((256,), jnp.float32),
        pltpu.VMEM((256,), jnp.float32),
        pltpu.VMEM((256,), jnp.float32),
        pltpu.VMEM((16,), jnp.float32),
        pltpu.VMEM((64,), jnp.float32),
        pltpu.SemaphoreType.DMA,
        pltpu.SemaphoreType.DMA,
        pltpu.SemaphoreType.DMA,
        pltpu.SemaphoreType.DMA,
    ],
    compiler_params=_params,
)
def _partials_kernel(pt_hbm, labels_hbm, out_hbm,
                     pbuf0, pbuf1, lbuf0, lbuf1, tbuf, tlbuf,
                     cnt_acc, conf_acc, acc_acc, bri_acc, partial,
                     psem0, psem1, lsem0, lsem1):
    wid = lax.axis_index("s") * NC + lax.axis_index("c")
    row0 = wid * BASE
    lane = lax.iota(jnp.int32, 16)
    zeros = jnp.zeros((16,), jnp.float32)
    ones = jnp.ones((16,), jnp.float32)

    for off in range(0, 256, 16):
        cnt_acc[pl.ds(off, 16)] = zeros
        conf_acc[pl.ds(off, 16)] = zeros
        acc_acc[pl.ds(off, 16)] = zeros
    bri_acc[...] = zeros

    def issue(pb, lb, ps, ls, ci):
        r0 = pl.multiple_of(row0 + ci * R, 128)
        pltpu.make_async_copy(pt_hbm.at[:, pl.ds(r0, R)], pb, ps).start()
        pltpu.make_async_copy(labels_hbm.at[pl.ds(r0, R)], lb, ls).start()

    def wait(pb, lb, ps, ls, ci):
        r0 = pl.multiple_of(row0 + ci * R, 128)
        pltpu.make_async_copy(pt_hbm.at[:, pl.ds(r0, R)], pb, ps).wait()
        pltpu.make_async_copy(labels_hbm.at[pl.ds(r0, R)], lb, ls).wait()

    def consume(pb, lb, ngroups):
        def group(o):
            rows = o + lane
            lab = lb[pl.ds(o, 16)]
            v = pb[0, pl.ds(o, 16)]
            s = v
            m = v
            sq = v * v
            for c in range(1, C):
                v = pb[c, pl.ds(o, 16)]
                s = s + v
                m = jnp.maximum(m, v)
                sq = sq + v * v
            sel = plsc.load_gather(pb, [lab, rows])
            inv = 1.0 / s
            conf = m * inv
            accv = (sel == m).astype(jnp.float32)
            bri_t = sq * inv * inv - 2.0 * sel * inv + 1.0
            b = (conf > LOWERS[1]).astype(jnp.int32)
            for k in range(2, 15):
                b = b + (conf > LOWERS[k]).astype(jnp.int32)
            return conf, accv, bri_t, lane * 16 + b

        def body(j, carry):
            o = pl.multiple_of(j * 64, 64)
            res = [group(o + 16 * u) for u in range(4)]
            bri_acc[...] = bri_acc[...] + ((res[0][2] + res[1][2])
                                           + (res[2][2] + res[3][2]))
            for conf_u, acc_u, _, idx_u in res:
                plsc.addupdate_scatter(cnt_acc, [idx_u], ones)
                plsc.addupdate_scatter(conf_acc, [idx_u], conf_u)
                plsc.addupdate_scatter(acc_acc, [idx_u], acc_u)
            return carry
        lax.fori_loop(0, ngroups // 4, body, 0)

    issue(pbuf0, lbuf0, psem0, lsem0, 0)

    def pair(j, carry):
        c0 = 2 * j
        issue(pbuf1, lbuf1, psem1, lsem1, c0 + 1)
        wait(pbuf0, lbuf0, psem0, lsem0, c0)
        consume(pbuf0, lbuf0, G)
        issue(pbuf0, lbuf0, psem0, lsem0, c0 + 2)
        wait(pbuf1, lbuf1, psem1, lsem1, c0 + 1)
        consume(pbuf1, lbuf1, G)
        return carry

    lax.fori_loop(0, NPAIR, pair, 0)
    wait(pbuf0, lbuf0, psem0, lsem0, NCH - 1)
    consume(pbuf0, lbuf0, G)

    if TAIL:
        @pl.when(wid < NTW)
        def _():
            t0 = pl.multiple_of(TSTART + wid * 128, 128)
            pltpu.make_async_copy(pt_hbm.at[:, pl.ds(t0, 128)], tbuf,
                                  psem1).start()
            pltpu.make_async_copy(labels_hbm.at[pl.ds(t0, 128)], tlbuf,
                                  lsem1).start()
            pltpu.make_async_copy(pt_hbm.at[:, pl.ds(t0, 128)], tbuf,
                                  psem1).wait()
            pltpu.make_async_copy(labels_hbm.at[pl.ds(t0, 128)], tlbuf,
                                  lsem1).wait()
            consume(tbuf, tlbuf, 8)

    cnt = cnt_acc[pl.ds(0, 16)]
    cf = conf_acc[pl.ds(0, 16)]
    ac = acc_acc[pl.ds(0, 16)]
    for l in range(1, 16):
        cnt = cnt + cnt_acc[pl.ds(16 * l, 16)]
        cf = cf + conf_acc[pl.ds(16 * l, 16)]
        ac = ac + acc_acc[pl.ds(16 * l, 16)]
    partial[pl.ds(0, 16)] = cnt
    partial[pl.ds(16, 16)] = cf
    partial[pl.ds(32, 16)] = ac
    partial[pl.ds(48, 16)] = bri_acc[...]
    off = pl.multiple_of(wid * 64, 8)
    pltpu.sync_copy(partial, out_hbm.at[pl.ds(off, 64)])


@functools.partial(
    pl.kernel,
    out_type=jax.ShapeDtypeStruct((16,), jnp.float32),
    mesh=_mesh,
    scratch_types=[
        pltpu.VMEM((NW * 64,), jnp.float32),
        pltpu.VMEM((16,), jnp.float32),
    ],
    compiler_params=_params,
)
def _finalize_kernel(partials_hbm, out_hbm, buf, obuf):
    wid = lax.axis_index("s") * NC + lax.axis_index("c")

    @pl.when(wid == 0)
    def _():
        pltpu.sync_copy(partials_hbm, buf)
        cnt = buf[pl.ds(0, 16)]
        cf = buf[pl.ds(16, 16)]
        ac = buf[pl.ds(32, 16)]
        bri = buf[pl.ds(48, 16)]
        for w in range(1, NW):
            o = w * 64
            cnt = cnt + buf[pl.ds(o, 16)]
            cf = cf + buf[pl.ds(o + 16, 16)]
            ac = ac + buf[pl.ds(o + 32, 16)]
            bri = bri + buf[pl.ds(o + 48, 16)]
        n = jnp.float32(N_ROWS)
        prop = cnt / n
        safe = jnp.maximum(cnt, 1.0)
        gap = jnp.abs(cf / safe - ac / safe)
        gapv = jnp.where(cnt > 0.0, gap, 0.0)
        ece = jnp.sum(gapv * prop)
        max_err = jnp.max(gapv)
        brim = jnp.sum(bri / n)
        lane = lax.iota(jnp.int32, 16)
        obuf[...] = (jnp.where(lane == 0, ece, 0.0)
                     + jnp.where(lane == 1, max_err, 0.0)
                     + jnp.where(lane == 2, brim, 0.0))
        pltpu.sync_copy(obuf, out_hbm)


def kernel(probs, labels):
    pt = probs.T
    lab = labels.astype(jnp.int32)
    partials = _partials_kernel(pt, lab)
    res = _finalize_kernel(partials)
    return (res[0:1], res[1:2], res[2])
